# baseline (speedup 1.0000x reference)
import functools

import jax
import jax.numpy as jnp
import numpy as np
from jax.experimental import pallas as pl

_HI = jax.lax.Precision.HIGHEST


def _conv_body(xp_ref, xc_ref, xn_ref, wm_ref, w0_ref, wp_ref, b_ref, o_ref,
               *, act, nbf, period):
    i = pl.program_id(0)
    cur = xc_ref[...]
    pr = xp_ref[nbf - 1:nbf, :]
    nx = xn_ref[0:1, :]
    prev_row = jnp.where(jax.lax.rem(i, period) == 0, jnp.zeros_like(pr), pr)
    next_row = jnp.where(jax.lax.rem(i, period) == period - 1,
                         jnp.zeros_like(nx), nx)
    xm = jnp.concatenate([prev_row, cur[:nbf - 1, :]], axis=0)
    xn = jnp.concatenate([cur[1:, :], next_row], axis=0)
    y = b_ref[...]
    for xx, w_ref in ((xm, wm_ref), (cur, w0_ref), (xn, wp_ref)):
        y = y + jax.lax.dot_general(
            xx.astype(jnp.bfloat16), w_ref[...].astype(jnp.bfloat16),
            (((1,), (0,)), ((), ())), preferred_element_type=jnp.float32)
    if act == "lrelu":
        y = jnp.where(y > 0, y, 0.1 * y)
    elif act == "tanh":
        y = jnp.tanh(y)
    o_ref[...] = y.astype(o_ref.dtype)


def _conv_mm(x3, ws, bias, act="none", nbf=2048, out_dtype=jnp.bfloat16):
    b, f, dd = x3.shape
    nbf = min(nbf, f)
    assert f % nbf == 0
    period = f // nbf
    n = b * f
    o = ws[0].shape[1]
    x = x3.reshape(n, dd)
    g = n // nbf
    specs = [
        pl.BlockSpec((nbf, dd), lambda i: (jnp.maximum(i - 1, 0), 0)),
        pl.BlockSpec((nbf, dd), lambda i: (i, 0)),
        pl.BlockSpec((nbf, dd), lambda i: (jnp.minimum(i + 1, g - 1), 0)),
    ]
    specs += [pl.BlockSpec(w.shape, lambda i: (0, 0)) for w in ws]
    specs += [pl.BlockSpec((1, o), lambda i: (0, 0))]
    return pl.pallas_call(
        functools.partial(_conv_body, act=act, nbf=nbf, period=period),
        grid=(g,),
        in_specs=specs,
        out_specs=pl.BlockSpec((nbf, o), lambda i: (i, 0)),
        out_shape=jax.ShapeDtypeStruct((n, o), out_dtype),
    )(x, x, x, *ws, bias)


def _vq_body(emb_ref, cb_ref, lw_ref, lb_ref, fw_ref, fb_ref, o_ref,
             *, n_batch, tf):
    emb = emb_ref[...]
    cb = cb_ref[...]
    kk, c = cb.shape
    n = emb.shape[0]
    s = jax.lax.dot_general(emb.astype(jnp.bfloat16), cb.astype(jnp.bfloat16),
                            (((1,), (1,)), ((), ())),
                            preferred_element_type=jnp.float32)
    cn2 = jax.lax.dot_general(jnp.full((1, c), 1.0, jnp.float32), cb * cb,
                              (((1,), (1,)), ((), ())),
                              precision=_HI,
                              preferred_element_type=jnp.float32)
    d = cn2 - 2.0 * s
    dmin = jnp.min(d, axis=1, keepdims=True)
    jidx = jax.lax.broadcasted_iota(jnp.int32, (n, kk), 1)
    codes = jnp.min(jnp.where(d == dmin, jidx, kk), axis=1, keepdims=True)
    onehot = (jidx == codes).astype(jnp.float32)
    emb_r = jax.lax.dot_general(onehot, cb, (((1,), (0,)), ((), ())),
                                precision=_HI,
                                preferred_element_type=jnp.float32)
    bidx = jax.lax.broadcasted_iota(jnp.int32, (n_batch, n), 1) // tf
    brow = jax.lax.broadcasted_iota(jnp.int32, (n_batch, n), 0)
    avg = jnp.where(bidx == brow, jnp.float32(1.0 / tf), 0.0)
    ta = jax.lax.dot_general(avg, emb, (((1,), (0,)), ((), ())),
                             precision=_HI,
                             preferred_element_type=jnp.float32)
    lin = jax.lax.dot_general(ta.astype(jnp.bfloat16),
                              lw_ref[...].astype(jnp.bfloat16),
                              (((1,), (1,)), ((), ())),
                              preferred_element_type=jnp.float32) + lb_ref[...]
    gb = jax.lax.dot_general(lin.astype(jnp.bfloat16),
                             fw_ref[...].astype(jnp.bfloat16),
                             (((1,), (1,)), ((), ())),
                             preferred_element_type=jnp.float32) + fb_ref[...]
    gamma = gb[:, :c]
    beta = gb[:, c:]
    for b in range(n_batch):
        lo, hi = b * tf, (b + 1) * tf
        o_ref[lo:hi, :] = (emb_r[lo:hi, :] * gamma[b:b + 1, :]
                           + beta[b:b + 1, :]).astype(o_ref.dtype)


def _vq_film(emb, cb, lw, lb, fw, fb):
    n, c = emb.shape
    n_batch, tf = 4, n // 4
    return pl.pallas_call(
        functools.partial(_vq_body, n_batch=n_batch, tf=tf),
        out_shape=jax.ShapeDtypeStruct((n, c), jnp.bfloat16),
    )(emb, cb, lw, lb[None, :], fw, fb[None, :])


def _enc_w(w, s, pad_l, offsets):
    o, c, _ = w.shape
    mats = []
    for d in offsets:
        m = jnp.zeros((s * c, o), jnp.float32)
        for j in range(s):
            k = s * d + j + pad_l
            if 0 <= k < 7:
                m = m.at[j * c:(j + 1) * c, :].set(w[:, :, k].T)
        mats.append(m)
    return mats


def _dec_w(w, s, pad_a, offsets):
    o, c, _ = w.shape
    mats = []
    for m in offsets:
        mat = jnp.zeros((c, s * o), jnp.float32)
        for p in range(s):
            k = s * m + pad_a - p
            if 0 <= k < 7:
                mat = mat.at[:, p * o:(p + 1) * o].set(w[:, :, k].T)
        mats.append(mat)
    return mats


def _e1_w(w1):
    mats = []
    for dshift in (-1, 0, 1):
        m = np.zeros((7, 128, 16, 4), np.float32)
        for a in range(16):
            for j in range(4):
                for k in range(7):
                    q = 8 * a + 2 * j + k - 2 - 128 * dshift
                    if 0 <= q < 128:
                        m[k, q, a, j] = 1.0
        mats.append(jnp.einsum('kqaj,ck->qajc', jnp.asarray(m),
                               w1[:, 0, :]).reshape(128, 4096))
    return mats


def _d4_w(w4):
    mats = []
    for dshift in (-1, 0, 1):
        m = np.zeros((7, 64, 64, 2), np.float32)
        for q in range(64):
            for j in range(64):
                for p in range(2):
                    k = 2 * (q + 64 * dshift - j) + 4 - p
                    if 0 <= k < 7:
                        m[k, q, j, p] = 1.0
        mats.append(jnp.einsum('kqjp,ck->qcjp', jnp.asarray(m),
                               w4[0]).reshape(4096, 128))
    return mats


def kernel(wav, enc_w1, enc_b1, enc_w2, enc_b2, enc_w3, enc_b3, enc_w4,
           enc_b4, codebook, lin_w, lin_b, film_w, film_b,
           dec_w1, dec_b1, dec_w2, dec_b2, dec_w3, dec_b3, dec_w4, dec_b4):
    B = wav.shape[0]
    T = wav.shape[2]

    x = wav.reshape(B, T // 128, 128)
    y = _conv_mm(x, _e1_w(enc_w1),
                 jnp.tile(enc_b1, 64)[None, :], "lrelu", nbf=256)

    x = y.reshape(B, T // 8, 4 * 64)
    y = _conv_mm(x, _enc_w(enc_w2, 4, 1, (-1, 0, 1)),
                 enc_b2[None, :], "lrelu", nbf=2048)

    x = y.reshape(B, T // 32, 4 * 128)
    y = _conv_mm(x, _enc_w(enc_w3, 4, 1, (-1, 0, 1)),
                 enc_b3[None, :], "lrelu", nbf=1024)

    x = y.reshape(B, T // 128, 4 * 256)
    emb = _conv_mm(x, _enc_w(enc_w4, 4, 1, (-1, 0, 1)),
                   enc_b4[None, :], "none", nbf=256,
                   out_dtype=jnp.float32)

    mod = _vq_film(emb, codebook, lin_w, lin_b, film_w, film_b)

    tf = T // 128
    x = mod.reshape(B, tf, 512)
    z = _conv_mm(x, _dec_w(dec_w1, 4, 5, (-1, 0, 1)),
                 jnp.tile(dec_b1, 4)[None, :], "lrelu", nbf=256)

    x = z.reshape(B, 4 * tf, 256)
    z = _conv_mm(x, _dec_w(dec_w2, 4, 5, (-1, 0, 1)),
                 jnp.tile(dec_b2, 4)[None, :], "lrelu", nbf=1024)

    x = z.reshape(B, 16 * tf, 128)
    z = _conv_mm(x, _dec_w(dec_w3, 4, 5, (-1, 0, 1)),
                 jnp.tile(dec_b3, 4)[None, :], "lrelu", nbf=2048)

    x = z.reshape(B, T // 128, 64 * 64)
    z = _conv_mm(x, _d4_w(dec_w4),
                 jnp.broadcast_to(dec_b4, (1, 128)), "tanh",
                 nbf=256, out_dtype=jnp.float32)

    return z.reshape(B, 1, T)
